# baseline (speedup 1.0000x reference)
import math

import jax
import jax.numpy as jnp
from jax.experimental import pallas as pl

N = 10000
H = 64
TEMB = 64
B = 8

_LOG1E4 = math.log(10000.0)


def _silu(v):
    return v * jax.nn.sigmoid(v)


def _mm(a, b):
    return jax.lax.dot_general(a, b, (((1,), (0,)), ((), ())),
                               preferred_element_type=jnp.float32)


def _body(x_ref, batch_ref, cin_ref, w128_ref, w64_ref, w32_ref, w16_ref,
          wnp2_ref, wpp2_ref, node_out_ref, pos_out_ref):

    half = TEMB // 2
    freq = jnp.exp(jax.lax.broadcasted_iota(jnp.int32, (1, half), 1)
                   .astype(jnp.float32) * (-_LOG1E4 / (half - 1)))
    ang = cin_ref[:, 0:1] * freq
    s, c = jnp.sin(ang), jnp.cos(ang)
    te_h = _silu(_mm(s, w128_ref[0:half, :]) + _mm(c, w128_ref[half:64, :])
                 + w128_ref[192:193, :])
    temb = _mm(te_h, w64_ref[0:128, :]) + w64_ref[460:461, :]

    te = _mm(_silu(_mm(cin_ref[:, 1:8], w32_ref[32:39, :]) + w32_ref[39:40, :]),
             w32_ref[0:32, :]) + w32_ref[40:41, :]
    se = _mm(_silu(_mm(cin_ref[:, 8:10], w16_ref[32:34, :])
                   + w16_ref[37:38, :]),
             w16_ref[0:16, :]) + w16_ref[38:39, :]
    ue = _mm(_silu(_mm(cin_ref[:, 10:13], w16_ref[34:37, :])
                   + w16_ref[39:40, :]),
             w16_ref[16:32, :]) + w16_ref[40:41, :]
    ce_h = _silu(_mm(te, w64_ref[256:288, :]) + _mm(se, w64_ref[288:304, :])
                 + _mm(ue, w64_ref[304:320, :]) + w64_ref[461:462, :])
    cond = _mm(ce_h, w64_ref[320:384, :]) + w64_ref[462:463, :]

    attn = (_mm(_mm(cond, w128_ref[64:128, :]), w64_ref[128:256, :])
            + w64_ref[463:464, :])
    delta = attn + _silu(temb)

    onehot = (batch_ref[...] ==
              jax.lax.broadcasted_iota(jnp.int32, (N, B), 1)
              ).astype(jnp.float32)
    u = _mm(onehot, delta)
    h = _mm(x_ref[...], w64_ref[448:460, :]) + w64_ref[464:465, :]
    h = h + u
    h = h + u
    h = h + u
    h = h + u
    a = _silu(_mm(h, w128_ref[128:192, :]) + w128_ref[193:194, :])
    node_out_ref[...] = _mm(a, wnp2_ref[0:128, :]) + wnp2_ref[128:129, :]
    g = _silu(_mm(h, w64_ref[384:448, :]) + w64_ref[465:466, :])
    pos_out_ref[...] = _mm(g, wpp2_ref[0:64, :]) + wpp2_ref[64:65, :]


def kernel(x, edge_index, edge_attr, pos, t, topo_cond, stab_cond, sust_cond,
           batch, params):
    del edge_index, edge_attr, pos
    p = params
    row = lambda b: b.reshape(1, -1)

    cin = jnp.concatenate([t.reshape(B, 1), topo_cond, stab_cond, sust_cond],
                          axis=1)
    w128 = jnp.concatenate([
        p['time1']['w'], p['Wv'], p['np1']['w'],
        row(p['time1']['b']), row(p['np1']['b'])], axis=0)
    w64 = jnp.concatenate([
        p['time2']['w'], p['Wo']['w'], p['comb1']['w'], p['comb2']['w'],
        p['pp1']['w'], p['node_emb']['w'],
        row(p['time2']['b']), row(p['comb1']['b']), row(p['comb2']['b']),
        row(p['Wo']['b']), row(p['node_emb']['b']), row(p['pp1']['b'])],
        axis=0)
    w32 = jnp.concatenate([
        p['topo2']['w'], p['topo1']['w'],
        row(p['topo1']['b']), row(p['topo2']['b'])], axis=0)
    w16 = jnp.concatenate([
        p['stab2']['w'], p['sust2']['w'], p['stab1']['w'], p['sust1']['w'],
        row(p['stab1']['b']), row(p['stab2']['b']),
        row(p['sust1']['b']), row(p['sust2']['b'])], axis=0)
    wnp2 = jnp.concatenate([p['np2']['w'], row(p['np2']['b'])], axis=0)
    wpp2 = jnp.concatenate([p['pp2']['w'], row(p['pp2']['b'])], axis=0)

    node_pred, pos_pred = pl.pallas_call(
        _body,
        out_shape=[
            jax.ShapeDtypeStruct((N, 12), jnp.float32),
            jax.ShapeDtypeStruct((N, 3), jnp.float32),
        ],
    )(x, batch.reshape(N, 1), cin, w128, w64, w32, w16, wnp2, wpp2)

    return node_pred, pos_pred


# speedup vs baseline: 1.5394x; 1.5394x over previous
import math

import jax
import jax.numpy as jnp
from jax.experimental import pallas as pl

N = 10000
H = 64
TEMB = 64
B = 8

_LOG1E4 = math.log(10000.0)


def _silu(v):
    return v * jax.nn.sigmoid(v)


def _mm(a, b):
    return jax.lax.dot_general(a, b, (((1,), (0,)), ((), ())),
                               preferred_element_type=jnp.float32)


def _body(x_ref, batch_ref, t_ref, topo_ref, stab_ref, sust_ref,
          t1w_ref, t1b_ref, t2w_ref, t2b_ref,
          to1w_ref, to1b_ref, to2w_ref, to2b_ref,
          st1w_ref, st1b_ref, st2w_ref, st2b_ref,
          su1w_ref, su1b_ref, su2w_ref, su2b_ref,
          c1w_ref, c1b_ref, c2w_ref, c2b_ref,
          wv_ref, wow_ref, wob_ref,
          new_ref, neb_ref,
          np1w_ref, np1b_ref, np2w_ref, np2b_ref,
          pp1w_ref, pp1b_ref, pp2w_ref, pp2b_ref,
          node_out_ref, pos_out_ref):
    half = TEMB // 2
    freq = jnp.exp(jax.lax.broadcasted_iota(jnp.int32, (1, half), 1)
                   .astype(jnp.float32) * (-_LOG1E4 / (half - 1)))
    ang = t_ref[...] * freq
    s, c = jnp.sin(ang), jnp.cos(ang)
    te_h = _silu(_mm(s, t1w_ref[:half, :]) + _mm(c, t1w_ref[half:, :])
                 + t1b_ref[...])
    temb = _mm(te_h, t2w_ref[...]) + t2b_ref[...]

    te = _mm(_silu(_mm(topo_ref[...], to1w_ref[...]) + to1b_ref[...]),
             to2w_ref[...]) + to2b_ref[...]
    se = _mm(_silu(_mm(stab_ref[...], st1w_ref[...]) + st1b_ref[...]),
             st2w_ref[...]) + st2b_ref[...]
    ue = _mm(_silu(_mm(sust_ref[...], su1w_ref[...]) + su1b_ref[...]),
             su2w_ref[...]) + su2b_ref[...]
    ce_h = _silu(_mm(te, c1w_ref[0:32, :]) + _mm(se, c1w_ref[32:48, :])
                 + _mm(ue, c1w_ref[48:64, :]) + c1b_ref[...])
    cond = _mm(ce_h, c2w_ref[...]) + c2b_ref[...]

    attn = _mm(_mm(cond, wv_ref[...]), wow_ref[...]) + wob_ref[...]
    delta = attn + _silu(temb)

    onehot = (batch_ref[...] ==
              jax.lax.broadcasted_iota(jnp.int32, (N, B), 1)
              ).astype(jnp.float32)
    u = _mm(onehot, delta)
    h = _mm(x_ref[...], new_ref[...]) + neb_ref[...]
    h = h + u
    h = h + u
    h = h + u
    h = h + u
    a = _silu(_mm(h, np1w_ref[...]) + np1b_ref[...])
    node_out_ref[...] = _mm(a, np2w_ref[...]) + np2b_ref[...]
    g = _silu(_mm(h, pp1w_ref[...]) + pp1b_ref[...])
    pos_out_ref[...] = _mm(g, pp2w_ref[...]) + pp2b_ref[...]


def kernel(x, edge_index, edge_attr, pos, t, topo_cond, stab_cond, sust_cond,
           batch, params):
    del edge_index, edge_attr, pos
    p = params
    row = lambda b: b.reshape(1, -1)

    node_pred, pos_pred = pl.pallas_call(
        _body,
        out_shape=[
            jax.ShapeDtypeStruct((N, 12), jnp.float32),
            jax.ShapeDtypeStruct((N, 3), jnp.float32),
        ],
    )(x, batch.reshape(N, 1), t.reshape(B, 1),
      topo_cond, stab_cond, sust_cond,
      p['time1']['w'], row(p['time1']['b']),
      p['time2']['w'], row(p['time2']['b']),
      p['topo1']['w'], row(p['topo1']['b']),
      p['topo2']['w'], row(p['topo2']['b']),
      p['stab1']['w'], row(p['stab1']['b']),
      p['stab2']['w'], row(p['stab2']['b']),
      p['sust1']['w'], row(p['sust1']['b']),
      p['sust2']['w'], row(p['sust2']['b']),
      p['comb1']['w'], row(p['comb1']['b']),
      p['comb2']['w'], row(p['comb2']['b']),
      p['Wv'], p['Wo']['w'], row(p['Wo']['b']),
      p['node_emb']['w'], row(p['node_emb']['b']),
      p['np1']['w'], row(p['np1']['b']),
      p['np2']['w'], row(p['np2']['b']),
      p['pp1']['w'], row(p['pp1']['b']),
      p['pp2']['w'], row(p['pp2']['b']))

    return node_pred, pos_pred
